# Initial kernel scaffold; baseline (speedup 1.0000x reference)
import functools

import jax
import jax.numpy as jnp
from jax import lax
from jax.experimental import pallas as pl
from jax.experimental.pallas import tpu as pltpu
from jax.experimental.pallas import tpu_sc as plsc

ROWS, COLS = 16384, 50
B = ROWS * COLS
D = 32

_info = plsc.get_sparse_core_info()
NC, NS = _info.num_cores, _info.num_subcores
NW = NC * NS
BPW = B // NW
CHUNK = 1280
NCHUNK = BPW // CHUNK

_mesh = plsc.VectorSubcoreMesh(core_axis_name="c", subcore_axis_name="s")


@functools.partial(
    pl.kernel,
    mesh=_mesh,
    out_type=jax.ShapeDtypeStruct((B, D), jnp.float32),
    scratch_types=[
        pltpu.VMEM((CHUNK,), jnp.int32),
        pltpu.VMEM((CHUNK, D), jnp.float32),
        pltpu.SemaphoreType.DMA,
    ],
)
def _gather(table_hbm, idx_hbm, out_hbm, idx_v, rows_v, sem):
    wid = lax.axis_index("s") * NC + lax.axis_index("c")
    base = wid * BPW

    def body(j, carry):
        off = base + j * CHUNK
        pltpu.sync_copy(idx_hbm.at[pl.ds(off, CHUNK)], idx_v)
        pltpu.async_copy(table_hbm.at[idx_v], rows_v, sem).wait()
        pltpu.sync_copy(rows_v, out_hbm.at[pl.ds(off, CHUNK)])
        return carry

    lax.fori_loop(0, NCHUNK, body, 0)


def kernel(x, table):
    flat = x.reshape(B)
    out = _gather(table, flat)
    return out.reshape(ROWS, COLS, D)


# speedup vs baseline: 1.0937x; 1.0937x over previous
import functools

import jax
import jax.numpy as jnp
from jax import lax
from jax.experimental import pallas as pl
from jax.experimental.pallas import tpu as pltpu
from jax.experimental.pallas import tpu_sc as plsc

ROWS, COLS = 16384, 50
B = ROWS * COLS
D = 32

_info = plsc.get_sparse_core_info()
NC, NS = _info.num_cores, _info.num_subcores
NW = NC * NS
BPW = B // NW
SEG = 128
NSEG = 8
CHUNK = SEG * NSEG
NCHUNK = BPW // CHUNK

_mesh = plsc.VectorSubcoreMesh(core_axis_name="c", subcore_axis_name="s")


@functools.partial(
    pl.kernel,
    mesh=_mesh,
    out_type=jax.ShapeDtypeStruct((B, D), jnp.float32),
    compiler_params=pltpu.CompilerParams(use_tc_tiling_on_sc=False),
    scratch_types=[
        pltpu.VMEM((NSEG, SEG), jnp.int32),
        pltpu.VMEM((CHUNK, D), jnp.float32),
        pltpu.SemaphoreType.DMA,
    ],
)
def _gather(table_hbm, idx_hbm, out_hbm, idx_v, rows_v, sem):
    wid = lax.axis_index("s") * NC + lax.axis_index("c")
    base = wid * BPW
    base_row = wid * (BPW // SEG)

    def body(j, carry):
        off = base + j * CHUNK
        pltpu.sync_copy(idx_hbm.at[pl.ds(base_row + j * NSEG, NSEG)], idx_v)
        copies = [
            pltpu.async_copy(
                table_hbm.at[idx_v.at[k]],
                rows_v.at[pl.ds(k * SEG, SEG)],
                sem,
            )
            for k in range(NSEG)
        ]
        for c in copies:
            c.wait()
        pltpu.sync_copy(rows_v, out_hbm.at[pl.ds(off, CHUNK)])
        return carry

    lax.fori_loop(0, NCHUNK, body, 0)


def kernel(x, table):
    flat = x.reshape(B // SEG, SEG)
    out = _gather(table, flat)
    return out.reshape(ROWS, COLS, D)


# speedup vs baseline: 1.1131x; 1.0177x over previous
import functools

import jax
import jax.numpy as jnp
from jax import lax
from jax.experimental import pallas as pl
from jax.experimental.pallas import tpu as pltpu
from jax.experimental.pallas import tpu_sc as plsc

ROWS, COLS = 16384, 50
B = ROWS * COLS
D = 32

_info = plsc.get_sparse_core_info()
NC, NS = _info.num_cores, _info.num_subcores
NW = NC * NS
BPW = B // NW
SEG = 128
NSEG = 10
CHUNK = SEG * NSEG
NCHUNK = BPW // CHUNK
IDXROWS = BPW // SEG

_mesh = plsc.VectorSubcoreMesh(core_axis_name="c", subcore_axis_name="s")


@functools.partial(
    pl.kernel,
    mesh=_mesh,
    out_type=jax.ShapeDtypeStruct((B, D), jnp.float32),
    compiler_params=pltpu.CompilerParams(use_tc_tiling_on_sc=False),
    scratch_types=[
        pltpu.VMEM((IDXROWS, SEG), jnp.int32),
        pltpu.VMEM((CHUNK, D), jnp.float32),
        pltpu.VMEM((CHUNK, D), jnp.float32),
        pltpu.SemaphoreType.DMA,
        pltpu.SemaphoreType.DMA,
        pltpu.SemaphoreType.DMA,
        pltpu.SemaphoreType.DMA,
    ],
)
def _gather(table_hbm, idx_hbm, out_hbm, idx_v, rows0, rows1, sg0, sg1, so0, so1):
    wid = lax.axis_index("s") * NC + lax.axis_index("c")
    base = wid * BPW
    pltpu.sync_copy(idx_hbm.at[pl.ds(wid * IDXROWS, IDXROWS)], idx_v)

    def fire_gathers(c, rows_b, sem):
        for k in range(NSEG):
            pltpu.async_copy(
                table_hbm.at[idx_v.at[c * NSEG + k]],
                rows_b.at[pl.ds(k * SEG, SEG)],
                sem,
            )

    def drain_gathers(rows_b, sem):
        for k in range(NSEG):
            pltpu.make_async_copy(
                table_hbm.at[idx_v.at[k]],
                rows_b.at[pl.ds(k * SEG, SEG)],
                sem,
            ).wait()

    def fire_store(c, rows_b, sem):
        pltpu.async_copy(rows_b, out_hbm.at[pl.ds(base + c * CHUNK, CHUNK)], sem)

    def wait_store(rows_b, sem):
        pltpu.make_async_copy(rows_b, out_hbm.at[pl.ds(base, CHUNK)], sem).wait()

    fire_gathers(0, rows0, sg0)
    fire_gathers(1, rows1, sg1)
    drain_gathers(rows0, sg0)
    fire_store(0, rows0, so0)

    def body(p, carry):
        ca = 2 * p + 1
        wait_store(rows0, so0)
        fire_gathers(ca + 1, rows0, sg0)
        drain_gathers(rows1, sg1)
        fire_store(ca, rows1, so1)

        cb = 2 * p + 2
        wait_store(rows1, so1)
        fire_gathers(cb + 1, rows1, sg1)
        drain_gathers(rows0, sg0)
        fire_store(cb, rows0, so0)
        return carry

    lax.fori_loop(0, (NCHUNK - 2) // 2, body, 0)

    wait_store(rows0, so0)
    drain_gathers(rows1, sg1)
    fire_store(NCHUNK - 1, rows1, so1)
    wait_store(rows1, so1)


def kernel(x, table):
    flat = x.reshape(B // SEG, SEG)
    out = _gather(table, flat)
    return out.reshape(ROWS, COLS, D)


# speedup vs baseline: 1.4639x; 1.3152x over previous
import functools

import jax
import jax.numpy as jnp
from jax import lax
from jax.experimental import pallas as pl
from jax.experimental.pallas import tpu as pltpu
from jax.experimental.pallas import tpu_sc as plsc

ROWS, COLS = 16384, 50
B = ROWS * COLS
D = 32

_info = plsc.get_sparse_core_info()
NC, NS, NL = _info.num_cores, _info.num_subcores, _info.num_lanes
NW = NC * NS
IW = ROWS // NW
SEG = 128
NSEG = IW // SEG
NBLK = IW // NL

_mesh = plsc.VectorSubcoreMesh(core_axis_name="c", subcore_axis_name="s")


@functools.partial(
    pl.kernel,
    mesh=_mesh,
    out_type=jax.ShapeDtypeStruct((COLS * D, ROWS), jnp.float32),
    compiler_params=pltpu.CompilerParams(
        use_tc_tiling_on_sc=False, needs_layout_passes=False
    ),
    scratch_types=[
        pltpu.VMEM((NSEG, SEG), jnp.int32),
        pltpu.VMEM((NSEG, SEG), jnp.int32),
        pltpu.VMEM((IW, D), jnp.float32),
        pltpu.VMEM((IW, D), jnp.float32),
        pltpu.VMEM((D, IW), jnp.float32),
        pltpu.VMEM((D, IW), jnp.float32),
        pltpu.SemaphoreType.DMA,
        pltpu.SemaphoreType.DMA,
        pltpu.SemaphoreType.DMA,
        pltpu.SemaphoreType.DMA,
    ],
)
def _gather(table_hbm, idx_hbm, out_hbm,
            idx0, idx1, rows0, rows1, tr0, tr1, sg0, sg1, so0, so1):
    wid = lax.axis_index("s") * NC + lax.axis_index("c")
    w4 = wid * NSEG
    col0 = wid * IW

    idx = (idx0, idx1)
    rows = (rows0, rows1)
    tr = (tr0, tr1)
    sg = (sg0, sg1)
    so = (so0, so1)

    def stage_in(j, b):
        pltpu.sync_copy(idx_hbm.at[pl.ds(j * (ROWS // SEG) + w4, NSEG)], idx[b])
        for k in range(NSEG):
            pltpu.async_copy(
                table_hbm.at[idx[b].at[k]],
                rows[b].at[pl.ds(k * SEG, SEG)],
                sg[b],
            )

    def drain_gathers(b):
        for k in range(NSEG):
            pltpu.make_async_copy(
                table_hbm.at[idx[b].at[k]],
                rows[b].at[pl.ds(k * SEG, SEG)],
                sg[b],
            ).wait()

    def transpose(b):
        def blk_body(blk, carry):
            base_i = blk * NL
            idx_i = base_i + lax.iota(jnp.int32, NL)
            for c in range(D):
                idx_c = jnp.full((NL,), c, jnp.int32)
                vals = plsc.load_gather(rows[b], [idx_i, idx_c])
                tr[b][c, pl.ds(base_i, NL)] = vals
            return carry

        lax.fori_loop(0, NBLK, blk_body, 0)

    def fire_store(j, b):
        pltpu.async_copy(
            tr[b], out_hbm.at[pl.ds(j * D, D), pl.ds(col0, IW)], so[b]
        )

    def wait_store(b):
        pltpu.make_async_copy(
            tr[b], out_hbm.at[pl.ds(0, D), pl.ds(col0, IW)], so[b]
        ).wait()

    def step(j, b, first=False, last=False):
        nb = 1 - b
        if not last:
            stage_in(j + 1, nb)
        drain_gathers(b)
        if not first:
            wait_store(b)
        transpose(b)
        fire_store(j, b)

    stage_in(0, 0)
    step(0, 0, first=True)
    step(1, 1, first=True)

    def body(p, carry):
        step(2 * p + 2, 0)
        step(2 * p + 3, 1)
        return carry

    lax.fori_loop(0, (COLS - 4) // 2, body, 0)

    step(COLS - 2, 0)
    step(COLS - 1, 1, last=True)
    wait_store(0)
    wait_store(1)


def kernel(x, table):
    idx2 = jnp.transpose(x).reshape(B // SEG, SEG)
    outp = _gather(table, idx2)
    return jnp.transpose(outp.reshape(COLS, D, ROWS), (2, 0, 1))


# speedup vs baseline: 1.9069x; 1.3026x over previous
import functools

import jax
import jax.numpy as jnp
from jax import lax
from jax.experimental import pallas as pl
from jax.experimental.pallas import tpu as pltpu
from jax.experimental.pallas import tpu_sc as plsc

ROWS, COLS = 16384, 50
B = ROWS * COLS
D = 32

_info = plsc.get_sparse_core_info()
NC, NS, NL = _info.num_cores, _info.num_subcores, _info.num_lanes
NW = NC * NS
IW = ROWS // NW
SEG = 128
NSEG = IW // SEG
NBLK = IW // NL

_mesh = plsc.VectorSubcoreMesh(core_axis_name="c", subcore_axis_name="s")


@functools.partial(
    pl.kernel,
    mesh=_mesh,
    out_type=jax.ShapeDtypeStruct((COLS * D, ROWS), jnp.float32),
    compiler_params=pltpu.CompilerParams(
        use_tc_tiling_on_sc=False, needs_layout_passes=False
    ),
    scratch_types=[
        pltpu.VMEM((NSEG, SEG), jnp.int32),
        pltpu.VMEM((NSEG, SEG), jnp.int32),
        pltpu.VMEM((IW, D), jnp.float32),
        pltpu.VMEM((IW, D), jnp.float32),
        pltpu.VMEM((D, IW), jnp.float32),
        pltpu.VMEM((D, IW), jnp.float32),
        pltpu.SemaphoreType.DMA,
        pltpu.SemaphoreType.DMA,
        pltpu.SemaphoreType.DMA,
        pltpu.SemaphoreType.DMA,
        pltpu.SemaphoreType.DMA,
        pltpu.SemaphoreType.DMA,
    ],
)
def _gather(table_hbm, idx_hbm, out_hbm,
            idx0, idx1, rows0, rows1, tr0, tr1,
            sg0, sg1, so0, so1, si0, si1):
    wid = lax.axis_index("s") * NC + lax.axis_index("c")
    w4 = wid * NSEG
    col0 = wid * IW

    idx = (idx0, idx1)
    rows = (rows0, rows1)
    tr = (tr0, tr1)
    sg = (sg0, sg1)
    so = (so0, so1)
    si = (si0, si1)

    def fire_idx(j, b):
        pltpu.async_copy(
            idx_hbm.at[pl.ds(j * (ROWS // SEG) + w4, NSEG)], idx[b], si[b]
        )

    def wait_idx(b):
        pltpu.make_async_copy(
            idx_hbm.at[pl.ds(w4, NSEG)], idx[b], si[b]
        ).wait()

    def fire_gathers(b):
        for k in range(NSEG):
            pltpu.async_copy(
                table_hbm.at[idx[b].at[k]],
                rows[b].at[pl.ds(k * SEG, SEG)],
                sg[b],
            )

    def drain_gathers(b):
        for k in range(NSEG):
            pltpu.make_async_copy(
                table_hbm.at[idx[b].at[k]],
                rows[b].at[pl.ds(k * SEG, SEG)],
                sg[b],
            ).wait()

    def transpose(b):
        def blk_body(blk, carry):
            base_i = blk * NL
            idx_i = base_i + lax.iota(jnp.int32, NL)
            vals = [
                plsc.load_gather(rows[b], [idx_i, jnp.full((NL,), c, jnp.int32)])
                for c in range(D)
            ]
            for c in range(D):
                tr[b][c, pl.ds(base_i, NL)] = vals[c]
            return carry

        lax.fori_loop(0, NBLK, blk_body, 0)

    def fire_store(j, b):
        pltpu.async_copy(
            tr[b], out_hbm.at[pl.ds(j * D, D), pl.ds(col0, IW)], so[b]
        )

    def wait_store(b):
        pltpu.make_async_copy(
            tr[b], out_hbm.at[pl.ds(0, D), pl.ds(col0, IW)], so[b]
        ).wait()

    def step(j, b, first=False, last=False, prefetch=True):
        nb = 1 - b
        if not last:
            wait_idx(nb)
            fire_gathers(nb)
        drain_gathers(b)
        if prefetch:
            fire_idx(j + 2, b)
        if not first:
            wait_store(b)
        transpose(b)
        fire_store(j, b)

    fire_idx(0, 0)
    wait_idx(0)
    fire_gathers(0)
    fire_idx(1, 1)
    step(0, 0, first=True)
    step(1, 1, first=True)

    def body(p, carry):
        step(2 * p + 2, 0)
        step(2 * p + 3, 1)
        return carry

    lax.fori_loop(0, (COLS - 4) // 2, body, 0)

    step(COLS - 2, 0, prefetch=False)
    step(COLS - 1, 1, last=True, prefetch=False)
    wait_store(0)
    wait_store(1)


def kernel(x, table):
    idx2 = jnp.transpose(x).reshape(B // SEG, SEG)
    outp = _gather(table, idx2)
    return jnp.transpose(outp.reshape(COLS, D, ROWS), (2, 0, 1))
